# baseline (speedup 1.0000x reference)
import functools

import jax
import jax.numpy as jnp
from jax import lax
from jax.experimental import pallas as pl
from jax.experimental.pallas import tpu as pltpu
from jax.experimental.pallas import tpu_sc as plsc

_EPS = 1e-12
_L = 16
_NBUF = 4


def _treesum(vs):
    while len(vs) > 1:
        vs = [a + b for a, b in zip(vs[::2], vs[1::2])]
    return vs[0]


def _build(B, S, H, V, P, CH):
    NV = H // _L
    NCH = S // CH
    mesh = plsc.VectorSubcoreMesh(core_axis_name="c", subcore_axis_name="s")

    vm = pltpu.VMEM
    f32 = jnp.float32
    assert CH == _L and P + 1 <= 2 * CH

    @functools.partial(
        pl.kernel,
        mesh=mesh,
        out_type=jax.ShapeDtypeStruct((B, S, H), f32),
        scratch_types=(
            [vm((S,), jnp.int32), vm((2 * CH,), jnp.int32)]
            + [vm((CH, H), f32) for _ in range(_NBUF)]
            + [vm((CH, H), f32), vm((CH, H), f32)]
            + [vm((2 * CH, H), f32)]
            + [vm((H,), f32), vm((H,), f32), vm((H,), f32)]
            + [pltpu.SMEM((CH,), f32), pltpu.SMEM((CH,), f32)]
            + [pltpu.SemaphoreType.DMA for _ in range(2 * _NBUF + 2)]
        ),
    )
    def emb_kernel(ids_hbm, word_hbm, pos_hbm, type_hbm, prompt_hbm,
                   gamma_hbm, beta_hbm, out_hbm, *refs):
        idx_all, pidx_v = refs[0:2]
        rows_b = refs[2:_NBUF + 2]
        pos_b = refs[_NBUF + 2:_NBUF + 4]
        prompt_v = refs[_NBUF + 4]
        type_v, gamma_v, beta_v = refs[_NBUF + 5:_NBUF + 8]
        a_sm, nma_sm = refs[_NBUF + 8:_NBUF + 10]
        sems = refs[_NBUF + 10:]
        gsems = sems[0:_NBUF]
        osems = sems[_NBUF:2 * _NBUF]
        psems = sems[2 * _NBUF:2 * _NBUF + 2]

        cid = lax.axis_index("c")
        sid = lax.axis_index("s")
        b = sid * 2 + cid
        lane = lax.broadcasted_iota(jnp.int32, (_L,), 0)

        pltpu.sync_copy(ids_hbm.at[b], idx_all)
        for v in range((1 + P + _L - 1) // _L):
            s_lo = v * _L
            msk = jnp.logical_and(lane + s_lo >= 1, lane + s_lo < 1 + P)
            cur = idx_all[pl.ds(s_lo, _L)]
            idx_all[pl.ds(s_lo, _L)] = jnp.where(msk, 0, cur)
        for v in range(2):
            pidx_v[pl.ds(v * _L, _L)] = jnp.minimum(lane + v * _L, P - 1)

        pltpu.sync_copy(type_hbm.at[0], type_v)
        pltpu.sync_copy(gamma_hbm, gamma_v)
        pltpu.sync_copy(beta_hbm, beta_v)
        pltpu.sync_copy(prompt_hbm.at[pidx_v], prompt_v)

        def start_gather(k, c):
            pltpu.async_copy(word_hbm.at[idx_all.at[pl.ds(c * CH, CH)]],
                             rows_b[k], gsems[k])

        def wait_gather(k, c):
            pltpu.make_async_copy(word_hbm.at[idx_all.at[pl.ds(c * CH, CH)]],
                                  rows_b[k], gsems[k]).wait()

        def start_pos(j, c):
            pltpu.async_copy(pos_hbm.at[pl.ds(c * CH, CH)], pos_b[j], psems[j])

        def wait_pos(j, c):
            pltpu.make_async_copy(pos_hbm.at[pl.ds(c * CH, CH)], pos_b[j],
                                  psems[j]).wait()

        def start_out(k, c):
            pltpu.async_copy(rows_b[k], out_hbm.at[b, pl.ds(c * CH, CH)],
                             osems[k])

        def wait_out(k, c):
            pltpu.make_async_copy(rows_b[k], out_hbm.at[b, pl.ds(c * CH, CH)],
                                  osems[k]).wait()

        def compute(k, c, j, prompt_part):
            rows = rows_b[k]
            pos = pos_b[j]
            if prompt_part == 0:
                @pl.when(c == 0)
                def _():
                    @plsc.parallel_loop(0, (CH - 1) * NV, unroll=4)
                    def pr_body(i):
                        r = i // NV
                        o = (i % NV) * _L
                        rows[r + 1, pl.ds(o, _L)] = (
                            rows[r + 1, pl.ds(o, _L)]
                            + prompt_v[r, pl.ds(o, _L)])
            elif prompt_part == 1:
                @pl.when(c == 1)
                def _():
                    @plsc.parallel_loop(0, (P - CH + 1) * NV, unroll=4)
                    def pr_body(i):
                        r = i // NV
                        o = (i % NV) * _L
                        rows[r, pl.ds(o, _L)] = (
                            rows[r, pl.ds(o, _L)]
                            + prompt_v[r + CH - 1, pl.ds(o, _L)])

            @plsc.parallel_loop(0, CH)
            def row_body(r):
                z = jnp.zeros((_L,), f32)
                accs = [z, z, z, z]
                acc2s = [z, z, z, z]
                for v in range(NV):
                    o = v * _L
                    x = (rows[r, pl.ds(o, _L)] + pos[r, pl.ds(o, _L)]
                         + type_v[pl.ds(o, _L)])
                    rows[r, pl.ds(o, _L)] = x
                    accs[v % 4] = accs[v % 4] + x
                    acc2s[v % 4] = acc2s[v % 4] + x * x
                acc = (accs[0] + accs[1]) + (accs[2] + accs[3])
                acc2 = (acc2s[0] + acc2s[1]) + (acc2s[2] + acc2s[3])
                s1 = _treesum([acc[jj] for jj in range(_L)])
                s2 = _treesum([acc2[jj] for jj in range(_L)])
                m = s1 * (1.0 / H)
                t = s2 * (1.0 / H) - m * m
                var = jnp.where(t > 0.0, t, 0.0) + _EPS
                iv = lax.bitcast_convert_type(var, jnp.int32)
                iv = jnp.int32(0x5F3759DF) - lax.shift_right_logical(iv, 1)
                y = lax.bitcast_convert_type(iv, f32)
                for _ in range(4):
                    y = y * (1.5 - 0.5 * var * y * y)
                a_sm[r] = y
                nma_sm[r] = -(m * y)

            def col_body(v, carry2):
                o = v * _L
                g = gamma_v[pl.ds(o, _L)]
                be = beta_v[pl.ds(o, _L)]

                @plsc.parallel_loop(0, CH, unroll=4)
                def p2_body(r):
                    x = rows[r, pl.ds(o, _L)]
                    xh = (x * jnp.full((_L,), a_sm[r])
                          + jnp.full((_L,), nma_sm[r]))
                    rows[r, pl.ds(o, _L)] = xh * g + be
                return carry2
            lax.fori_loop(0, NV, col_body, 0)

        start_gather(0, 0)
        start_gather(1, 1)
        start_pos(0, 0)

        def quad_body(t, carry):
            for k in range(_NBUF):
                c = t * _NBUF + k
                k2 = (k + 2) % _NBUF
                j = k % 2

                @pl.when(c + 2 < NCH)
                def _():
                    @pl.when(c >= 2)
                    def _():
                        wait_out(k2, c - 2)
                    start_gather(k2, c + 2)

                @pl.when(c + 1 < NCH)
                def _():
                    start_pos(1 - j, c + 1)

                wait_gather(k, c)
                wait_pos(j, c)
                compute(k, c, j, prompt_part=(k if k < 2 else -1))
                start_out(k, c)
            return carry

        lax.fori_loop(0, NCH // _NBUF, quad_body, 0)
        for k in range(_NBUF):
            wait_out(k, NCH - _NBUF + k)

    return emb_kernel


@jax.jit
def kernel(input_ids, word_emb, pos_emb, type_emb, prompt_emb, gamma, beta):
    B, S = input_ids.shape
    V, H = word_emb.shape
    P = prompt_emb.shape[0]
    emb = _build(B, S, H, V, P, CH=16)
    return emb(input_ids, word_emb, pos_emb, type_emb, prompt_emb, gamma, beta)
